# baseline (speedup 1.0000x reference)
import functools
import math

import jax
import jax.numpy as jnp
from jax import lax
from jax.experimental import pallas as pl
from jax.experimental.pallas import tpu as pltpu
from jax.experimental.pallas import tpu_sc as plsc

F = 256
L = 20
RC = 5.0
EPS = 1e-8
LOG2 = math.log(2.0)

N = 10000
E = 160000

FW = 32
NC1 = 8
CB = 128
NC2 = 12
WTC = 160
T1C = 256
CNTC = 128
KB = 80
NT = 16
NSC = 2

EPT = E // NT
KB1 = 40
NB1 = EPT // KB1
NBS2 = EPT // KB
FT = 10
RPF = N // FT
ZR = 40
FLB = 200

EPC = E // NSC
EPT2 = EPC // NT
KB2 = 40
NB2 = EPT2 // KB2


def _ssp(x):
    return jax.nn.softplus(x) - LOG2


def _ka_body(s_ref, vt_ref, w1_ref, b1_ref, w2_ref, b2_ref, t1_ref):
    s = s_ref[...]
    h = _ssp(s @ w1_ref[...] + b1_ref[...])
    phi = h @ w2_ref[...] + b2_ref[...]
    phi_v = phi[:, :F]
    phi_s = phi[:, F:2 * F]
    phi_r = phi[:, 2 * F:]
    a = vt_ref[...] * jnp.concatenate([phi_v, phi_v, phi_v], axis=1)
    for c in range(NC1):
        fr = slice(c * FW, (c + 1) * FW)
        t1_ref[c, :, 0:FW] = phi_s[:, fr]
        t1_ref[c, :, FW:2 * FW] = phi_r[:, fr]
        for cc in range(3):
            t1_ref[c, :, (2 + cc) * FW:(3 + cc) * FW] = a[:, cc * F + c * FW: cc * F + (c + 1) * FW]
        t1_ref[c, :, 5 * FW:T1C] = jnp.zeros((s.shape[0], T1C - 5 * FW), jnp.float32)


def _kw_body(ev_ref, mvw_ref, mvb_ref, wtab_ref):
    ev = ev_ref[...]
    r = jnp.sqrt(jnp.sum(ev * ev, axis=1, keepdims=True) + EPS)
    n = (lax.broadcasted_iota(jnp.int32, (1, L), 1) + 1).astype(jnp.float32)
    rbf = jnp.sin(n * (jnp.pi / RC) * r) / r
    fc = jnp.where(r < RC, 0.5 * (jnp.cos(jnp.pi * r / RC) + 1.0), 0.0)
    w = fc * (rbf @ mvw_ref[...] + mvb_ref[...])
    u = ev / r
    for c in range(NC1):
        fr = slice(c * FW, (c + 1) * FW)
        wr = w[:, 2 * F + c * FW: 2 * F + (c + 1) * FW]
        wtab_ref[c, :, 0:FW] = w[:, F + c * FW: F + (c + 1) * FW]
        for cc in range(3):
            wtab_ref[c, :, (1 + cc) * FW:(2 + cc) * FW] = wr * u[:, cc:cc + 1]
        wtab_ref[c, :, 4 * FW:WTC] = w[:, fr]


def _kd_body(o0_ref, o1_ref, o2_ref, o3_ref, cp_ref, s0_ref, vt0_ref,
             u1w_ref, u1b_ref, u2w_ref, u2b_ref, s_ref, vt_ref, t2_ref):
    cp = cp_ref[...]
    cnt = cp[0, :, 0:1] + cp[1, :, 0:1]
    cntc = jnp.maximum(cnt, 1.0)
    parts = [o0_ref[...], o1_ref[...], o2_ref[...], o3_ref[...]]
    ds = jnp.concatenate([parts[c // 2][c % 2, :, 0:FW] for c in range(NC1)], axis=1)
    dvt = jnp.concatenate(
        [parts[c // 2][c % 2, :, (1 + cc) * FW:(2 + cc) * FW]
         for cc in range(3) for c in range(NC1)], axis=1)
    s = s0_ref[...] + ds / cntc
    vt = vt0_ref[...] + dvt / cntc
    v0 = vt[:, :F]
    v1 = vt[:, F:2 * F]
    v2 = vt[:, 2 * F:]
    norm = jnp.sqrt(v0 * v0 + v1 * v1 + v2 * v2 + EPS)
    h = _ssp(jnp.concatenate([norm, s], axis=1) @ u1w_ref[...] + u1b_ref[...])
    g = h @ u2w_ref[...] + u2b_ref[...]
    s_ref[...] = s
    vt_ref[...] = vt
    h2 = jnp.concatenate([vt, g], axis=1)
    for t in range(NC2):
        t2_ref[t, :, :] = h2[:, t * CB:(t + 1) * CB]


def _ke_body(s_ref, vt_ref, q0_ref, q1_ref, q2_ref, q3_ref, q4_ref, q5_ref,
             cp_ref, outs_ref, outvt_ref):
    cp = cp_ref[...]
    cnt = cp[0, :, 0:1] + cp[1, :, 0:1]
    cntc = jnp.maximum(cnt, 1.0)
    qs = [q0_ref[...], q1_ref[...], q2_ref[...], q3_ref[...], q4_ref[...], q5_ref[...]]
    sumv = jnp.concatenate([qs[t // 2][t % 2] for t in range(6)], axis=1)
    sumg = jnp.concatenate([qs[3 + t // 2][t % 2] for t in range(6)], axis=1)
    uvt = sumv / cntc
    sagg = sumg / cntc
    u0 = uvt[:, :F]
    u1 = uvt[:, F:2 * F]
    u2 = uvt[:, 2 * F:]
    q2n = u0 * u0 + u1 * u1 + u2 * u2 + EPS
    avv = sagg[:, :F]
    asv = sagg[:, F:2 * F]
    ass = sagg[:, 2 * F:]
    ds2 = ((q2n - EPS) / q2n) * asv + ass
    outs_ref[...] = s_ref[...] + ds2
    outvt_ref[...] = vt_ref[...] + uvt * jnp.concatenate([avv, avv, avv], axis=1)


_MESH = plsc.VectorSubcoreMesh(core_axis_name="c", subcore_axis_name="s")


def _s1_all():
    @functools.partial(
        pl.kernel, mesh=_MESH,
        out_type=jax.ShapeDtypeStruct((5 * NSC * N, CB), jnp.float32),
        scratch_types=[
            pltpu.VMEM((KB1,), jnp.int32),
            pltpu.VMEM((KB1,), jnp.int32),
            pltpu.VMEM((KB1,), jnp.int32),
            pltpu.VMEM((KB1,), jnp.int32),
            pltpu.VMEM((KB1, T1C), jnp.float32),
            pltpu.VMEM((KB1, T1C), jnp.float32),
            pltpu.VMEM((KB1, WTC), jnp.float32),
            pltpu.VMEM((KB1, WTC), jnp.float32),
            pltpu.VMEM((ZR, CB), jnp.float32),
            pltpu.VMEM_SHARED((N, CB), jnp.float32),
            pltpu.SemaphoreType.DMA,
            pltpu.SemaphoreType.DMA,
        ],
    )
    def body(t1_hbm, wtab_hbm, src_hbm, dst_hbm, out_hbm,
             idxda, idxsa, idxdb, idxsb, rowsa, rowsb, wrowa, wrowb,
             zc, acc, sema, semb):
        core = lax.axis_index("c")
        sub = lax.axis_index("s")
        zv = jnp.zeros((16,), jnp.float32)
        r0 = sub * RPF
        tb = sub * EPT

        def zero_acc():
            def zrow(i, _):
                for m in range(CB // 16):
                    zc[i, pl.ds(m * 16, 16)] = zv
                return 0
            lax.fori_loop(0, ZR, zrow, 0)

            @pl.when(sub < FT)
            def _zero():
                for t in range(RPF // ZR):
                    pltpu.sync_copy(zc, acc.at[pl.ds(r0 + t * ZR, ZR)])
            plsc.subcore_barrier()

        def flush(obase):
            plsc.subcore_barrier()

            @pl.when(sub < FT)
            def _flush():
                for t in range(RPF // FLB):
                    pltpu.sync_copy(acc.at[pl.ds(r0 + t * FLB, FLB)],
                                    out_hbm.at[pl.ds(obase + r0 + t * FLB, FLB)])

        for p in range(4):
            cidx = 2 * p + core
            off = cidx * N
            zero_acc()

            def issue(b, idxd, idxs, rows, wrow, sem):
                e0 = tb + b * KB1
                pltpu.sync_copy(dst_hbm.at[pl.ds(e0, KB1)], idxd)
                pltpu.sync_copy(src_hbm.at[pl.ds(e0, KB1)], idxs)
                for m in range(KB1 // 16):
                    idxd[pl.ds(m * 16, 16)] = idxd[pl.ds(m * 16, 16)] + off
                if KB1 % 16:
                    tail = jnp.where(
                        lax.iota(jnp.int32, 16) < 16 - KB1 % 16, 0, off)
                    idxd[pl.ds(KB1 - 16, 16)] = idxd[pl.ds(KB1 - 16, 16)] + tail
                pltpu.async_copy(t1_hbm.at[idxd], rows, sem)
                pltpu.async_copy(wtab_hbm.at[pl.ds(cidx * E + e0, KB1)], wrow, sem)

            def consume(idxd, idxs, rows, wrow, sem):
                pltpu.make_async_copy(t1_hbm.at[idxd], rows, sem).wait()
                pltpu.make_async_copy(wtab_hbm.at[pl.ds(0, KB1)], wrow, sem).wait()

                def edge(i, _):
                    for j2 in range(2):
                        o = j2 * 16
                        cv = rows[i, pl.ds(o, 16)]
                        bv = rows[i, pl.ds(FW + o, 16)]
                        av0 = rows[i, pl.ds(2 * FW + o, 16)]
                        av1 = rows[i, pl.ds(3 * FW + o, 16)]
                        av2 = rows[i, pl.ds(4 * FW + o, 16)]
                        ws = wrow[i, pl.ds(o, 16)]
                        wru0 = wrow[i, pl.ds(FW + o, 16)]
                        wru1 = wrow[i, pl.ds(2 * FW + o, 16)]
                        wru2 = wrow[i, pl.ds(3 * FW + o, 16)]
                        wv = wrow[i, pl.ds(4 * FW + o, 16)]
                        zc[i, pl.ds(o, 16)] = cv * ws
                        zc[i, pl.ds(FW + o, 16)] = av0 * wv + bv * wru0
                        zc[i, pl.ds(2 * FW + o, 16)] = av1 * wv + bv * wru1
                        zc[i, pl.ds(3 * FW + o, 16)] = av2 * wv + bv * wru2
                    return 0

                def edge4(i, _):
                    for e4 in range(4):
                        edge(i * 4 + e4, 0)
                    return 0
                lax.fori_loop(0, KB1 // 4, edge4, 0)
                pltpu.sync_copy(zc, acc.at[idxs], add=True)

            issue(0, idxda, idxsa, rowsa, wrowa, sema)

            def pair(g, _):
                issue(2 * g + 1, idxdb, idxsb, rowsb, wrowb, semb)
                consume(idxda, idxsa, rowsa, wrowa, sema)

                @pl.when(2 * g + 2 < NB1)
                def _():
                    issue(2 * g + 2, idxda, idxsa, rowsa, wrowa, sema)
                consume(idxdb, idxsb, rowsb, wrowb, semb)
                return 0
            lax.fori_loop(0, NB1 // 2, pair, 0)
            flush((2 * p + core) * N)

        zero_acc()
        li = lax.iota(jnp.int32, 16)
        onev = jnp.where(li == 0, 1.0, 0.0).astype(jnp.float32)

        def orow(i, _):
            ones_row = onev
            zc[i, pl.ds(0, 16)] = ones_row
            for m in range(1, CB // 16):
                zc[i, pl.ds(m * 16, 16)] = zv
            return 0
        lax.fori_loop(0, KB2, orow, 0)
        tb2 = core * EPC + sub * EPT2

        def cbatch(b, _):
            e0 = tb2 + b * KB2
            pltpu.sync_copy(src_hbm.at[pl.ds(e0, KB2)], idxsa)
            pltpu.sync_copy(zc, acc.at[idxsa], add=True)
            return 0
        lax.fori_loop(0, NB2, cbatch, 0)
        flush((8 + core) * N)
    return body


def _s2_all():
    @functools.partial(
        pl.kernel, mesh=_MESH,
        out_type=jax.ShapeDtypeStruct((6 * NSC * N, CB), jnp.float32),
        scratch_types=[
            pltpu.VMEM((KB,), jnp.int32),
            pltpu.VMEM((KB,), jnp.int32),
            pltpu.VMEM((KB,), jnp.int32),
            pltpu.VMEM((KB,), jnp.int32),
            pltpu.VMEM((KB, CB), jnp.float32),
            pltpu.VMEM((KB, CB), jnp.float32),
            pltpu.VMEM((ZR, CB), jnp.float32),
            pltpu.VMEM_SHARED((N, CB), jnp.float32),
            pltpu.SemaphoreType.DMA,
            pltpu.SemaphoreType.DMA,
        ],
    )
    def body(t2_hbm, src_hbm, dst_hbm, out_hbm,
             idxda, idxsa, idxdb, idxsb, rowsa, rowsb, zbuf, acc, sema, semb):
        core = lax.axis_index("c")
        sub = lax.axis_index("s")
        zv = jnp.zeros((16,), jnp.float32)
        r0 = sub * RPF
        tb = sub * EPT

        for p in range(6):
            cidx = 2 * p + core
            off = cidx * N

            def zrow(i, _):
                for m in range(CB // 16):
                    zbuf[i, pl.ds(m * 16, 16)] = zv
                return 0
            lax.fori_loop(0, ZR, zrow, 0)

            @pl.when(sub < FT)
            def _zero():
                for t in range(RPF // ZR):
                    pltpu.sync_copy(zbuf, acc.at[pl.ds(r0 + t * ZR, ZR)])
            plsc.subcore_barrier()

            def issue(b, idxd, idxs, rows, sem):
                e0 = tb + b * KB
                pltpu.sync_copy(dst_hbm.at[pl.ds(e0, KB)], idxd)
                pltpu.sync_copy(src_hbm.at[pl.ds(e0, KB)], idxs)
                for m in range(KB // 16):
                    idxd[pl.ds(m * 16, 16)] = idxd[pl.ds(m * 16, 16)] + off
                pltpu.async_copy(t2_hbm.at[idxd], rows, sem)

            def consume(idxd, idxs, rows, sem):
                pltpu.make_async_copy(t2_hbm.at[idxd], rows, sem).wait()
                pltpu.sync_copy(rows, acc.at[idxs], add=True)

            issue(0, idxda, idxsa, rowsa, sema)

            def pair(g, _):
                issue(2 * g + 1, idxdb, idxsb, rowsb, semb)
                consume(idxda, idxsa, rowsa, sema)
                issue(2 * g + 2, idxda, idxsa, rowsa, sema)
                consume(idxdb, idxsb, rowsb, semb)
                return 0
            lax.fori_loop(0, NBS2 // 2, pair, 0)
            consume(idxda, idxsa, rowsa, sema)
            plsc.subcore_barrier()

            @pl.when(sub < FT)
            def _flush():
                for t in range(RPF // FLB):
                    pltpu.sync_copy(
                        acc.at[pl.ds(r0 + t * FLB, FLB)],
                        out_hbm.at[pl.ds((2 * p + core) * N + r0 + t * FLB, FLB)])
    return body


_NBK = 1000
_EBK = 2000


def _full(shape):
    return pl.BlockSpec(shape, lambda i: tuple(0 for _ in shape))


def kernel(node_fea_s, node_fea_v, edge_index, edge_attr, batch, edge_vec,
           ms1_w, ms1_b, ms2_w, ms2_b, mv_w, mv_b, us1_w, us1_b, us2_w, us2_b):
    src = edge_index[0]
    dst = edge_index[1]
    vt0 = node_fea_v.transpose(0, 2, 1).reshape(N, 3 * F)

    t1 = pl.pallas_call(
        _ka_body,
        grid=(N // _NBK,),
        in_specs=[
            pl.BlockSpec((_NBK, F), lambda i: (i, 0)),
            pl.BlockSpec((_NBK, 3 * F), lambda i: (i, 0)),
            _full((F, F)),
            _full((1, F)),
            _full((F, 3 * F)),
            _full((1, 3 * F)),
        ],
        out_specs=pl.BlockSpec((NC1, _NBK, T1C), lambda i: (0, i, 0)),
        out_shape=jax.ShapeDtypeStruct((NC1, N, T1C), jnp.float32),
    )(node_fea_s, vt0, ms1_w.T, ms1_b.reshape(1, -1), ms2_w.T, ms2_b.reshape(1, -1))

    wtab = pl.pallas_call(
        _kw_body,
        grid=(E // _EBK,),
        in_specs=[
            pl.BlockSpec((_EBK, 3), lambda i: (i, 0)),
            _full((L, 3 * F)),
            _full((1, 3 * F)),
        ],
        out_specs=pl.BlockSpec((NC1, _EBK, WTC), lambda i: (0, i, 0)),
        out_shape=jax.ShapeDtypeStruct((NC1, E, WTC), jnp.float32),
    )(edge_vec, mv_w.T, mv_b.reshape(1, -1))

    t1f = t1.reshape(NC1 * N, T1C)
    wtabf = wtab.reshape(NC1 * E, WTC)

    o1all = _s1_all()(t1f, wtabf, src, dst).reshape(5, NSC, N, CB)
    o1 = [o1all[j] for j in range(4)]
    cp = o1all[4]

    s, vt, t2 = pl.pallas_call(
        _kd_body,
        grid=(N // _NBK,),
        in_specs=[
            pl.BlockSpec((NSC, _NBK, CB), lambda i: (0, i, 0)),
            pl.BlockSpec((NSC, _NBK, CB), lambda i: (0, i, 0)),
            pl.BlockSpec((NSC, _NBK, CB), lambda i: (0, i, 0)),
            pl.BlockSpec((NSC, _NBK, CB), lambda i: (0, i, 0)),
            pl.BlockSpec((NSC, _NBK, CNTC), lambda i: (0, i, 0)),
            pl.BlockSpec((_NBK, F), lambda i: (i, 0)),
            pl.BlockSpec((_NBK, 3 * F), lambda i: (i, 0)),
            _full((2 * F, F)),
            _full((1, F)),
            _full((F, 3 * F)),
            _full((1, 3 * F)),
        ],
        out_specs=[
            pl.BlockSpec((_NBK, F), lambda i: (i, 0)),
            pl.BlockSpec((_NBK, 3 * F), lambda i: (i, 0)),
            pl.BlockSpec((NC2, _NBK, CB), lambda i: (0, i, 0)),
        ],
        out_shape=[
            jax.ShapeDtypeStruct((N, F), jnp.float32),
            jax.ShapeDtypeStruct((N, 3 * F), jnp.float32),
            jax.ShapeDtypeStruct((NC2, N, CB), jnp.float32),
        ],
    )(o1[0], o1[1], o1[2], o1[3], cp, node_fea_s, vt0,
      us1_w.T, us1_b.reshape(1, -1), us2_w.T, us2_b.reshape(1, -1))

    t2f = t2.reshape(NC2 * N, CB)

    o2all = _s2_all()(t2f, src, dst).reshape(6, NSC, N, CB)
    o2 = [o2all[j] for j in range(6)]

    outs, outvt = pl.pallas_call(
        _ke_body,
        grid=(N // _NBK,),
        in_specs=[
            pl.BlockSpec((_NBK, F), lambda i: (i, 0)),
            pl.BlockSpec((_NBK, 3 * F), lambda i: (i, 0)),
        ] + [pl.BlockSpec((NSC, _NBK, CB), lambda i: (0, i, 0)) for _ in range(6)] + [
            pl.BlockSpec((NSC, _NBK, CNTC), lambda i: (0, i, 0)),
        ],
        out_specs=[
            pl.BlockSpec((_NBK, F), lambda i: (i, 0)),
            pl.BlockSpec((_NBK, 3 * F), lambda i: (i, 0)),
        ],
        out_shape=[
            jax.ShapeDtypeStruct((N, F), jnp.float32),
            jax.ShapeDtypeStruct((N, 3 * F), jnp.float32),
        ],
    )(s, vt, o2[0], o2[1], o2[2], o2[3], o2[4], o2[5], cp)

    out_v = outvt.reshape(N, 3, F).transpose(0, 2, 1)
    return (outs, out_v)


# speedup vs baseline: 1.1498x; 1.1498x over previous
import functools
import math

import jax
import jax.numpy as jnp
from jax import lax
from jax.experimental import pallas as pl
from jax.experimental.pallas import tpu as pltpu
from jax.experimental.pallas import tpu_sc as plsc

F = 256
L = 20
RC = 5.0
EPS = 1e-8
LOG2 = math.log(2.0)

N = 10000
E = 160000

FW = 32
NC1 = 8
CB = 128
NC2 = 12
WTC = 160
T1C = 256
CNTC = 128
KB = 80
NT = 16
NSC = 2

EPT = E // NT
KB1 = 40
NB1 = EPT // KB1
NBS2 = EPT // KB
FT = 10
RPF = N // FT
ZR = 40
FLB = 200

EPC = E // NSC
EPT2 = EPC // NT
KB2 = 40
NB2 = EPT2 // KB2


def _ssp(x):
    return jax.nn.softplus(x) - LOG2


def _ka_body(s_ref, vt_ref, w1_ref, b1_ref, w2_ref, b2_ref, t1_ref):
    s = s_ref[...]
    h = _ssp(s @ w1_ref[...] + b1_ref[...])
    phi = h @ w2_ref[...] + b2_ref[...]
    phi_v = phi[:, :F]
    phi_s = phi[:, F:2 * F]
    phi_r = phi[:, 2 * F:]
    a = vt_ref[...] * jnp.concatenate([phi_v, phi_v, phi_v], axis=1)
    for c in range(NC1):
        fr = slice(c * FW, (c + 1) * FW)
        t1_ref[c, :, 0:FW] = phi_s[:, fr]
        t1_ref[c, :, FW:2 * FW] = phi_r[:, fr]
        for cc in range(3):
            t1_ref[c, :, (2 + cc) * FW:(3 + cc) * FW] = a[:, cc * F + c * FW: cc * F + (c + 1) * FW]
        t1_ref[c, :, 5 * FW:T1C] = jnp.zeros((s.shape[0], T1C - 5 * FW), jnp.float32)


def _kw_body(ev_ref, mvw_ref, mvb_ref, wtab_ref):
    ev = ev_ref[...]
    r = jnp.sqrt(jnp.sum(ev * ev, axis=1, keepdims=True) + EPS)
    n = (lax.broadcasted_iota(jnp.int32, (1, L), 1) + 1).astype(jnp.float32)
    rbf = jnp.sin(n * (jnp.pi / RC) * r) / r
    fc = jnp.where(r < RC, 0.5 * (jnp.cos(jnp.pi * r / RC) + 1.0), 0.0)
    w = fc * (rbf @ mvw_ref[...] + mvb_ref[...])
    u = ev / r
    for c in range(NC1):
        fr = slice(c * FW, (c + 1) * FW)
        wr = w[:, 2 * F + c * FW: 2 * F + (c + 1) * FW]
        wtab_ref[c, :, 0:FW] = w[:, F + c * FW: F + (c + 1) * FW]
        for cc in range(3):
            wtab_ref[c, :, (1 + cc) * FW:(2 + cc) * FW] = wr * u[:, cc:cc + 1]
        wtab_ref[c, :, 4 * FW:WTC] = w[:, fr]


def _kd_body(o0_ref, o1_ref, o2_ref, o3_ref, cp_ref, s0_ref, vt0_ref,
             u1w_ref, u1b_ref, u2w_ref, u2b_ref, s_ref, vt_ref, t2_ref):
    cp = cp_ref[...]
    cnt = cp[0, :, 0:1] + cp[1, :, 0:1]
    cntc = jnp.maximum(cnt, 1.0)
    parts = [o0_ref[...], o1_ref[...], o2_ref[...], o3_ref[...]]
    ds = jnp.concatenate([parts[c // 2][c % 2, :, 0:FW] for c in range(NC1)], axis=1)
    dvt = jnp.concatenate(
        [parts[c // 2][c % 2, :, (1 + cc) * FW:(2 + cc) * FW]
         for cc in range(3) for c in range(NC1)], axis=1)
    s = s0_ref[...] + ds / cntc
    vt = vt0_ref[...] + dvt / cntc
    v0 = vt[:, :F]
    v1 = vt[:, F:2 * F]
    v2 = vt[:, 2 * F:]
    norm = jnp.sqrt(v0 * v0 + v1 * v1 + v2 * v2 + EPS)
    h = _ssp(jnp.concatenate([norm, s], axis=1) @ u1w_ref[...] + u1b_ref[...])
    g = h @ u2w_ref[...] + u2b_ref[...]
    s_ref[...] = s
    vt_ref[...] = vt
    h2 = jnp.concatenate([vt, g], axis=1)
    for t in range(NC2):
        t2_ref[t, :, :] = h2[:, t * CB:(t + 1) * CB]


def _ke_body(s_ref, vt_ref, q0_ref, q1_ref, q2_ref, q3_ref, q4_ref, q5_ref,
             cp_ref, outs_ref, outvt_ref):
    cp = cp_ref[...]
    cnt = cp[0, :, 0:1] + cp[1, :, 0:1]
    cntc = jnp.maximum(cnt, 1.0)
    qs = [q0_ref[...], q1_ref[...], q2_ref[...], q3_ref[...], q4_ref[...], q5_ref[...]]
    sumv = jnp.concatenate([qs[t // 2][t % 2] for t in range(6)], axis=1)
    sumg = jnp.concatenate([qs[3 + t // 2][t % 2] for t in range(6)], axis=1)
    uvt = sumv / cntc
    sagg = sumg / cntc
    u0 = uvt[:, :F]
    u1 = uvt[:, F:2 * F]
    u2 = uvt[:, 2 * F:]
    q2n = u0 * u0 + u1 * u1 + u2 * u2 + EPS
    avv = sagg[:, :F]
    asv = sagg[:, F:2 * F]
    ass = sagg[:, 2 * F:]
    ds2 = ((q2n - EPS) / q2n) * asv + ass
    outs_ref[...] = s_ref[...] + ds2
    outvt_ref[...] = vt_ref[...] + uvt * jnp.concatenate([avv, avv, avv], axis=1)


_MESH = plsc.VectorSubcoreMesh(core_axis_name="c", subcore_axis_name="s")


def _s1_all():
    @functools.partial(
        pl.kernel, mesh=_MESH,
        out_type=jax.ShapeDtypeStruct((5 * NSC * N, CB), jnp.float32),
        scratch_types=[
            pltpu.VMEM((KB1,), jnp.int32),
            pltpu.VMEM((KB1,), jnp.int32),
            pltpu.VMEM((KB1,), jnp.int32),
            pltpu.VMEM((KB1,), jnp.int32),
            pltpu.VMEM((KB1, T1C), jnp.float32),
            pltpu.VMEM((KB1, T1C), jnp.float32),
            pltpu.VMEM((KB1, WTC), jnp.float32),
            pltpu.VMEM((KB1, WTC), jnp.float32),
            pltpu.VMEM((ZR, CB), jnp.float32),
            pltpu.VMEM_SHARED((N, CB), jnp.float32),
            pltpu.SemaphoreType.DMA,
            pltpu.SemaphoreType.DMA,
        ],
    )
    def body(t1_hbm, wtab_hbm, src_hbm, dst_hbm, out_hbm,
             idxda, idxsa, idxdb, idxsb, rowsa, rowsb, wrowa, wrowb,
             zc, acc, sema, semb):
        core = lax.axis_index("c")
        sub = lax.axis_index("s")
        zv = jnp.zeros((16,), jnp.float32)
        r0 = sub * RPF
        tb = sub * EPT

        def zero_acc():
            def zrow(i, _):
                for m in range(CB // 16):
                    zc[i, pl.ds(m * 16, 16)] = zv
                return 0
            lax.fori_loop(0, ZR, zrow, 0)

            @pl.when(sub < FT)
            def _zero():
                for t in range(RPF // ZR):
                    pltpu.sync_copy(zc, acc.at[pl.ds(r0 + t * ZR, ZR)])
            plsc.subcore_barrier()

        def flush(obase):
            plsc.subcore_barrier()

            @pl.when(sub < FT)
            def _flush():
                for t in range(RPF // FLB):
                    pltpu.sync_copy(acc.at[pl.ds(r0 + t * FLB, FLB)],
                                    out_hbm.at[pl.ds(obase + r0 + t * FLB, FLB)])

        for p in range(4):
            cidx = 2 * p + core
            off = cidx * N
            zero_acc()

            def issue(b, idxd, idxs, rows, wrow, sem):
                e0 = tb + b * KB1
                pltpu.sync_copy(dst_hbm.at[pl.ds(e0, KB1)], idxd)
                pltpu.sync_copy(src_hbm.at[pl.ds(e0, KB1)], idxs)
                for m in range(KB1 // 16):
                    idxd[pl.ds(m * 16, 16)] = idxd[pl.ds(m * 16, 16)] + off
                if KB1 % 16:
                    tail = jnp.where(
                        lax.iota(jnp.int32, 16) < 16 - KB1 % 16, 0, off)
                    idxd[pl.ds(KB1 - 16, 16)] = idxd[pl.ds(KB1 - 16, 16)] + tail
                pltpu.async_copy(t1_hbm.at[idxd], rows, sem)
                pltpu.async_copy(wtab_hbm.at[pl.ds(cidx * E + e0, KB1)], wrow, sem)

            def consume(idxd, idxs, rows, wrow, sem):
                pltpu.make_async_copy(t1_hbm.at[idxd], rows, sem).wait()
                pltpu.make_async_copy(wtab_hbm.at[pl.ds(0, KB1)], wrow, sem).wait()

                @functools.partial(plsc.parallel_loop, 0, KB1, unroll=4)
                def _edges(i):
                    for j2 in range(2):
                        o = j2 * 16
                        cv = rows[i, pl.ds(o, 16)]
                        bv = rows[i, pl.ds(FW + o, 16)]
                        av0 = rows[i, pl.ds(2 * FW + o, 16)]
                        av1 = rows[i, pl.ds(3 * FW + o, 16)]
                        av2 = rows[i, pl.ds(4 * FW + o, 16)]
                        ws = wrow[i, pl.ds(o, 16)]
                        wru0 = wrow[i, pl.ds(FW + o, 16)]
                        wru1 = wrow[i, pl.ds(2 * FW + o, 16)]
                        wru2 = wrow[i, pl.ds(3 * FW + o, 16)]
                        wv = wrow[i, pl.ds(4 * FW + o, 16)]
                        zc[i, pl.ds(o, 16)] = cv * ws
                        zc[i, pl.ds(FW + o, 16)] = av0 * wv + bv * wru0
                        zc[i, pl.ds(2 * FW + o, 16)] = av1 * wv + bv * wru1
                        zc[i, pl.ds(3 * FW + o, 16)] = av2 * wv + bv * wru2
                pltpu.sync_copy(zc, acc.at[idxs], add=True)

            issue(0, idxda, idxsa, rowsa, wrowa, sema)

            def pair(g, _):
                issue(2 * g + 1, idxdb, idxsb, rowsb, wrowb, semb)
                consume(idxda, idxsa, rowsa, wrowa, sema)

                @pl.when(2 * g + 2 < NB1)
                def _():
                    issue(2 * g + 2, idxda, idxsa, rowsa, wrowa, sema)
                consume(idxdb, idxsb, rowsb, wrowb, semb)
                return 0
            lax.fori_loop(0, NB1 // 2, pair, 0)
            flush((2 * p + core) * N)

        zero_acc()
        li = lax.iota(jnp.int32, 16)
        onev = jnp.where(li == 0, 1.0, 0.0).astype(jnp.float32)

        def orow(i, _):
            ones_row = onev
            zc[i, pl.ds(0, 16)] = ones_row
            for m in range(1, CB // 16):
                zc[i, pl.ds(m * 16, 16)] = zv
            return 0
        lax.fori_loop(0, KB2, orow, 0)
        tb2 = core * EPC + sub * EPT2

        def cbatch(b, _):
            e0 = tb2 + b * KB2
            pltpu.sync_copy(src_hbm.at[pl.ds(e0, KB2)], idxsa)
            pltpu.sync_copy(zc, acc.at[idxsa], add=True)
            return 0
        lax.fori_loop(0, NB2, cbatch, 0)
        flush((8 + core) * N)
    return body


def _s2_all():
    @functools.partial(
        pl.kernel, mesh=_MESH,
        out_type=jax.ShapeDtypeStruct((6 * NSC * N, CB), jnp.float32),
        scratch_types=[
            pltpu.VMEM((KB,), jnp.int32),
            pltpu.VMEM((KB,), jnp.int32),
            pltpu.VMEM((KB,), jnp.int32),
            pltpu.VMEM((KB,), jnp.int32),
            pltpu.VMEM((KB, CB), jnp.float32),
            pltpu.VMEM((KB, CB), jnp.float32),
            pltpu.VMEM((ZR, CB), jnp.float32),
            pltpu.VMEM_SHARED((N, CB), jnp.float32),
            pltpu.SemaphoreType.DMA,
            pltpu.SemaphoreType.DMA,
        ],
    )
    def body(t2_hbm, src_hbm, dst_hbm, out_hbm,
             idxda, idxsa, idxdb, idxsb, rowsa, rowsb, zbuf, acc, sema, semb):
        core = lax.axis_index("c")
        sub = lax.axis_index("s")
        zv = jnp.zeros((16,), jnp.float32)
        r0 = sub * RPF
        tb = sub * EPT

        for p in range(6):
            cidx = 2 * p + core
            off = cidx * N

            def zrow(i, _):
                for m in range(CB // 16):
                    zbuf[i, pl.ds(m * 16, 16)] = zv
                return 0
            lax.fori_loop(0, ZR, zrow, 0)

            @pl.when(sub < FT)
            def _zero():
                for t in range(RPF // ZR):
                    pltpu.sync_copy(zbuf, acc.at[pl.ds(r0 + t * ZR, ZR)])
            plsc.subcore_barrier()

            def issue(b, idxd, idxs, rows, sem):
                e0 = tb + b * KB
                pltpu.sync_copy(dst_hbm.at[pl.ds(e0, KB)], idxd)
                pltpu.sync_copy(src_hbm.at[pl.ds(e0, KB)], idxs)
                for m in range(KB // 16):
                    idxd[pl.ds(m * 16, 16)] = idxd[pl.ds(m * 16, 16)] + off
                pltpu.async_copy(t2_hbm.at[idxd], rows, sem)

            def consume(idxd, idxs, rows, sem):
                pltpu.make_async_copy(t2_hbm.at[idxd], rows, sem).wait()
                pltpu.sync_copy(rows, acc.at[idxs], add=True)

            issue(0, idxda, idxsa, rowsa, sema)

            def pair(g, _):
                issue(2 * g + 1, idxdb, idxsb, rowsb, semb)
                consume(idxda, idxsa, rowsa, sema)
                issue(2 * g + 2, idxda, idxsa, rowsa, sema)
                consume(idxdb, idxsb, rowsb, semb)
                return 0
            lax.fori_loop(0, NBS2 // 2, pair, 0)
            consume(idxda, idxsa, rowsa, sema)
            plsc.subcore_barrier()

            @pl.when(sub < FT)
            def _flush():
                for t in range(RPF // FLB):
                    pltpu.sync_copy(
                        acc.at[pl.ds(r0 + t * FLB, FLB)],
                        out_hbm.at[pl.ds((2 * p + core) * N + r0 + t * FLB, FLB)])
    return body


_NBK = 1000
_EBK = 2000


def _full(shape):
    return pl.BlockSpec(shape, lambda i: tuple(0 for _ in shape))


def kernel(node_fea_s, node_fea_v, edge_index, edge_attr, batch, edge_vec,
           ms1_w, ms1_b, ms2_w, ms2_b, mv_w, mv_b, us1_w, us1_b, us2_w, us2_b):
    src = edge_index[0]
    dst = edge_index[1]
    vt0 = node_fea_v.transpose(0, 2, 1).reshape(N, 3 * F)

    t1 = pl.pallas_call(
        _ka_body,
        grid=(N // _NBK,),
        in_specs=[
            pl.BlockSpec((_NBK, F), lambda i: (i, 0)),
            pl.BlockSpec((_NBK, 3 * F), lambda i: (i, 0)),
            _full((F, F)),
            _full((1, F)),
            _full((F, 3 * F)),
            _full((1, 3 * F)),
        ],
        out_specs=pl.BlockSpec((NC1, _NBK, T1C), lambda i: (0, i, 0)),
        out_shape=jax.ShapeDtypeStruct((NC1, N, T1C), jnp.float32),
    )(node_fea_s, vt0, ms1_w.T, ms1_b.reshape(1, -1), ms2_w.T, ms2_b.reshape(1, -1))

    wtab = pl.pallas_call(
        _kw_body,
        grid=(E // _EBK,),
        in_specs=[
            pl.BlockSpec((_EBK, 3), lambda i: (i, 0)),
            _full((L, 3 * F)),
            _full((1, 3 * F)),
        ],
        out_specs=pl.BlockSpec((NC1, _EBK, WTC), lambda i: (0, i, 0)),
        out_shape=jax.ShapeDtypeStruct((NC1, E, WTC), jnp.float32),
    )(edge_vec, mv_w.T, mv_b.reshape(1, -1))

    t1f = t1.reshape(NC1 * N, T1C)
    wtabf = wtab.reshape(NC1 * E, WTC)

    o1all = _s1_all()(t1f, wtabf, src, dst).reshape(5, NSC, N, CB)
    o1 = [o1all[j] for j in range(4)]
    cp = o1all[4]

    s, vt, t2 = pl.pallas_call(
        _kd_body,
        grid=(N // _NBK,),
        in_specs=[
            pl.BlockSpec((NSC, _NBK, CB), lambda i: (0, i, 0)),
            pl.BlockSpec((NSC, _NBK, CB), lambda i: (0, i, 0)),
            pl.BlockSpec((NSC, _NBK, CB), lambda i: (0, i, 0)),
            pl.BlockSpec((NSC, _NBK, CB), lambda i: (0, i, 0)),
            pl.BlockSpec((NSC, _NBK, CNTC), lambda i: (0, i, 0)),
            pl.BlockSpec((_NBK, F), lambda i: (i, 0)),
            pl.BlockSpec((_NBK, 3 * F), lambda i: (i, 0)),
            _full((2 * F, F)),
            _full((1, F)),
            _full((F, 3 * F)),
            _full((1, 3 * F)),
        ],
        out_specs=[
            pl.BlockSpec((_NBK, F), lambda i: (i, 0)),
            pl.BlockSpec((_NBK, 3 * F), lambda i: (i, 0)),
            pl.BlockSpec((NC2, _NBK, CB), lambda i: (0, i, 0)),
        ],
        out_shape=[
            jax.ShapeDtypeStruct((N, F), jnp.float32),
            jax.ShapeDtypeStruct((N, 3 * F), jnp.float32),
            jax.ShapeDtypeStruct((NC2, N, CB), jnp.float32),
        ],
    )(o1[0], o1[1], o1[2], o1[3], cp, node_fea_s, vt0,
      us1_w.T, us1_b.reshape(1, -1), us2_w.T, us2_b.reshape(1, -1))

    t2f = t2.reshape(NC2 * N, CB)

    o2all = _s2_all()(t2f, src, dst).reshape(6, NSC, N, CB)
    o2 = [o2all[j] for j in range(6)]

    outs, outvt = pl.pallas_call(
        _ke_body,
        grid=(N // _NBK,),
        in_specs=[
            pl.BlockSpec((_NBK, F), lambda i: (i, 0)),
            pl.BlockSpec((_NBK, 3 * F), lambda i: (i, 0)),
        ] + [pl.BlockSpec((NSC, _NBK, CB), lambda i: (0, i, 0)) for _ in range(6)] + [
            pl.BlockSpec((NSC, _NBK, CNTC), lambda i: (0, i, 0)),
        ],
        out_specs=[
            pl.BlockSpec((_NBK, F), lambda i: (i, 0)),
            pl.BlockSpec((_NBK, 3 * F), lambda i: (i, 0)),
        ],
        out_shape=[
            jax.ShapeDtypeStruct((N, F), jnp.float32),
            jax.ShapeDtypeStruct((N, 3 * F), jnp.float32),
        ],
    )(s, vt, o2[0], o2[1], o2[2], o2[3], o2[4], o2[5], cp)

    out_v = outvt.reshape(N, 3, F).transpose(0, 2, 1)
    return (outs, out_v)
